# baseline (speedup 1.0000x reference)
import functools

import jax
import jax.numpy as jnp
from jax import lax
from jax.experimental import pallas as pl
from jax.experimental.pallas import tpu as pltpu
from jax.experimental.pallas import tpu_sc as plsc

B, L = 4096, 200
HID = 128
NFIELDS = 26
MLP_DIM = 512
VPAD = 32
NW = 32
EPC = 32
GC0, GC1 = 104, 96
ACC = NFIELDS * HID


def _sc_pool_kernel(nb, index_hbm, fv_hbm, emb_hbm,
                    xe_hbm,
                    idx_0, fv_0, idx_1, fv_1,
                    rows_a, rows_b, acc_a, acc_b,
                    sem_i0, sem_i1, sem_ga, sem_gb, sem_oa, sem_ob):
    nc = 2
    wid = lax.axis_index("s") * nc + lax.axis_index("c")
    n_per_w = nb // NW
    base = wid * n_per_w

    zeros16f = jnp.zeros((16,), jnp.float32)
    iota16 = lax.iota(jnp.int32, 16)

    def g_start(idx_v, e, rows_ref, sem):
        off = e * L
        pltpu.make_async_copy(
            emb_hbm.at[idx_v.at[pl.ds(off, GC0)]],
            rows_ref.at[pl.ds(0, GC0)], sem).start()
        pltpu.make_async_copy(
            emb_hbm.at[idx_v.at[pl.ds(off + GC0, GC1)]],
            rows_ref.at[pl.ds(GC0, GC1)], sem).start()

    def g_wait(idx_v, e, rows_ref, sem):
        off = e * L
        pltpu.make_async_copy(
            emb_hbm.at[idx_v.at[pl.ds(off, GC0)]],
            rows_ref.at[pl.ds(0, GC0)], sem).wait()
        pltpu.make_async_copy(
            emb_hbm.at[idx_v.at[pl.ds(off + GC0, GC1)]],
            rows_ref.at[pl.ds(GC0, GC1)], sem).wait()

    def wb_start(b, acc_ref, sem):
        pltpu.make_async_copy(acc_ref, xe_hbm.at[pl.ds(b * ACC, ACC)], sem).start()

    def wb_wait(b, acc_ref, sem):
        pltpu.make_async_copy(acc_ref, xe_hbm.at[pl.ds(b * ACC, ACC)], sem).wait()

    def compute(fv_v, e, rows_ref, acc_ref):
        @plsc.parallel_loop(0, ACC // 16, unroll=8)
        def _(i):
            acc_ref[pl.ds(16 * i, 16)] = zeros16f

        off = e * L

        @plsc.parallel_loop(0, L, unroll=4)
        def _(l):
            pos = jnp.full((16,), off + l, jnp.int32)
            fv = plsc.load_gather(fv_v, [pos])
            fs = fv & 31
            vs = lax.bitcast_convert_type(fv & ~31, jnp.float32)
            addr = fs * HID + iota16
            for j in range(HID // 16):
                x = vs * rows_ref[l, pl.ds(16 * j, 16)]
                plsc.addupdate_scatter(acc_ref, [addr + 16 * j], x)

    bufs = ((idx_0, fv_0, sem_i0), (idx_1, fv_1, sem_i1))

    def in_copies(c, bufset):
        idx_v, fv_v, sem_in = bufset
        cb = base + c * EPC
        return (
            pltpu.make_async_copy(
                index_hbm.at[pl.ds(cb * L, EPC * L)], idx_v, sem_in),
            pltpu.make_async_copy(
                fv_hbm.at[pl.ds(cb * L, EPC * L)], fv_v, sem_in),
        )

    n_chunks = n_per_w // EPC
    for cp in in_copies(0, bufs[0]):
        cp.start()

    for c in range(n_chunks):
        idx_v, fv_v, sem_in = bufs[c % 2]
        cb = base + c * EPC
        for cp in in_copies(c, bufs[c % 2]):
            cp.wait()
        if c + 1 < n_chunks:
            for cp in in_copies(c + 1, bufs[(c + 1) % 2]):
                cp.start()

        g_start(idx_v, 0, rows_a, sem_ga)

        def pair(ep, carry):
            e0 = 2 * ep
            e1 = 2 * ep + 1
            g_start(idx_v, e1, rows_b, sem_gb)

            @pl.when(ep > 0)
            def _():
                wb_wait(cb + e0 - 2, acc_a, sem_oa)
            g_wait(idx_v, e0, rows_a, sem_ga)
            compute(fv_v, e0, rows_a, acc_a)
            wb_start(cb + e0, acc_a, sem_oa)

            @pl.when(ep < EPC // 2 - 1)
            def _():
                g_start(idx_v, e1 + 1, rows_a, sem_ga)

            @pl.when(ep > 0)
            def _():
                wb_wait(cb + e1 - 2, acc_b, sem_ob)
            g_wait(idx_v, e1, rows_b, sem_gb)
            compute(fv_v, e1, rows_b, acc_b)
            wb_start(cb + e1, acc_b, sem_ob)
            return carry

        lax.fori_loop(0, EPC // 2, pair, 0)
        wb_wait(cb + EPC - 2, acc_a, sem_oa)
        wb_wait(cb + EPC - 1, acc_b, sem_ob)


def _sc_pool(index, fv, emb, nb):
    mesh = plsc.VectorSubcoreMesh(core_axis_name="c", subcore_axis_name="s")
    kern = pl.kernel(
        functools.partial(_sc_pool_kernel, nb),
        mesh=mesh,
        compiler_params=pltpu.CompilerParams(needs_layout_passes=False),
        out_type=jax.ShapeDtypeStruct((nb * ACC,), jnp.float32),
        scratch_types=[
            pltpu.VMEM((EPC * L,), jnp.int32),
            pltpu.VMEM((EPC * L,), jnp.int32),
            pltpu.VMEM((EPC * L,), jnp.int32),
            pltpu.VMEM((EPC * L,), jnp.int32),
            pltpu.VMEM((L, HID), jnp.float32),
            pltpu.VMEM((L, HID), jnp.float32),
            pltpu.VMEM((ACC,), jnp.float32),
            pltpu.VMEM((ACC,), jnp.float32),
            pltpu.SemaphoreType.DMA,
            pltpu.SemaphoreType.DMA,
            pltpu.SemaphoreType.DMA,
            pltpu.SemaphoreType.DMA,
            pltpu.SemaphoreType.DMA,
            pltpu.SemaphoreType.DMA,
        ],
    )
    return kern(index, fv, emb)


def _m_kernel(fe_ref, w1f_ref, m_ref):
    for s in range(NFIELDS):
        m_ref[s, :] = jnp.dot(fe_ref[s, :].reshape(1, HID), w1f_ref[s],
                              preferred_element_type=jnp.float32)[0]


def _mlp_kernel(xe_ref, val_ref, fld_ref, w1e_ref, m_ref, b1_ref, w2_ref,
                b2_ref, out_ref):
    h = jnp.dot(xe_ref[...].astype(jnp.bfloat16),
                w1e_ref[...].astype(jnp.bfloat16),
                preferred_element_type=jnp.float32)
    cols = []
    for s in range(VPAD):
        masked = jnp.where(fld_ref[...] == s, val_ref[...], 0.0)
        cols.append(jnp.sum(masked, axis=1, keepdims=True))
    vs = jnp.concatenate(cols, axis=1)
    h = h + jnp.dot(vs, m_ref[...], preferred_element_type=jnp.float32)
    h = jax.nn.relu(h + b1_ref[...])
    y = jnp.dot(h, w2_ref[...], preferred_element_type=jnp.float32) + b2_ref[0, 0]
    out_ref[...] = y


def kernel(index, value, field, emb, field_emb, W1, b1, W2, b2):
    index = index.astype(jnp.int32)
    field = field.astype(jnp.int32)

    vbits = lax.bitcast_convert_type(value, jnp.int32)
    fv = (vbits & ~31) | field
    index_f = index.reshape(-1)
    fv_f = fv.reshape(-1)

    w1r = W1.reshape(NFIELDS, 2, HID, MLP_DIM)
    w1e = w1r[:, 0].reshape(NFIELDS * HID, MLP_DIM)
    w1f = w1r[:, 1]
    fe = field_emb[:NFIELDS]

    m = pl.pallas_call(
        _m_kernel,
        out_shape=jax.ShapeDtypeStruct((NFIELDS, MLP_DIM), jnp.float32),
    )(fe, w1f)
    m_pad = jnp.pad(m, ((0, VPAD - NFIELDS), (0, 0)))

    nh = 4
    nb = B // nh
    bt = 256
    outs = []
    for h in range(nh):
        xe_h = _sc_pool(index_f[h * nb * L:(h + 1) * nb * L],
                        fv_f[h * nb * L:(h + 1) * nb * L], emb, nb)
        tile0 = h * (nb // bt)
        out_h = pl.pallas_call(
            _mlp_kernel,
            grid=(nb // bt,),
            in_specs=[
                pl.BlockSpec((bt, NFIELDS * HID), lambda i: (i, 0)),
                pl.BlockSpec((bt, L), lambda i, t=tile0: (i + t, 0)),
                pl.BlockSpec((bt, L), lambda i, t=tile0: (i + t, 0)),
                pl.BlockSpec((NFIELDS * HID, MLP_DIM), lambda i: (0, 0)),
                pl.BlockSpec((VPAD, MLP_DIM), lambda i: (0, 0)),
                pl.BlockSpec((1, MLP_DIM), lambda i: (0, 0)),
                pl.BlockSpec((MLP_DIM, 1), lambda i: (0, 0)),
                pl.BlockSpec((1, 1), lambda i: (0, 0)),
            ],
            out_specs=pl.BlockSpec((bt, 1), lambda i: (i, 0)),
            out_shape=jax.ShapeDtypeStruct((nb, 1), jnp.float32),
        )(xe_h.reshape(nb, NFIELDS * HID), value, field, w1e, m_pad,
          b1.reshape(1, MLP_DIM), W2, b2.reshape(1, 1))
        outs.append(out_h[:, 0])

    return jnp.concatenate(outs)


# speedup vs baseline: 1.0104x; 1.0104x over previous
import functools

import jax
import jax.numpy as jnp
from jax import lax
from jax.experimental import pallas as pl
from jax.experimental.pallas import tpu as pltpu
from jax.experimental.pallas import tpu_sc as plsc

B, L = 4096, 200
VOCAB = 100001
HID = 128
NFIELDS = 26
MLP_DIM = 512
VPAD = 32
NW = 32
EPC = 32
GC0, GC1 = 104, 96
ACC = NFIELDS * HID


def _sc_pool_kernel(nb, index_hbm, fv_hbm, emb_hbm,
                    xe_hbm,
                    idx_0, fv_0, idx_1, fv_1,
                    rows_a, rows_b, acc_a, acc_b,
                    sem_i0, sem_i1, sem_ga, sem_gb, sem_oa, sem_ob):
    nc = 2
    wid = lax.axis_index("s") * nc + lax.axis_index("c")
    n_per_w = nb // NW
    base = wid * n_per_w

    zeros16f = jnp.zeros((16,), jnp.float32)
    iota16 = lax.iota(jnp.int32, 16)

    def g_start(idx_v, e, rows_ref, sem):
        off = e * L
        pltpu.make_async_copy(
            emb_hbm.at[idx_v.at[pl.ds(off, GC0)]],
            rows_ref.at[pl.ds(0, GC0)], sem).start()
        pltpu.make_async_copy(
            emb_hbm.at[idx_v.at[pl.ds(off + GC0, GC1)]],
            rows_ref.at[pl.ds(GC0, GC1)], sem).start()

    def g_wait(idx_v, e, rows_ref, sem):
        off = e * L
        pltpu.make_async_copy(
            emb_hbm.at[idx_v.at[pl.ds(off, GC0)]],
            rows_ref.at[pl.ds(0, GC0)], sem).wait()
        pltpu.make_async_copy(
            emb_hbm.at[idx_v.at[pl.ds(off + GC0, GC1)]],
            rows_ref.at[pl.ds(GC0, GC1)], sem).wait()

    def wb_start(b, acc_ref, sem):
        pltpu.make_async_copy(acc_ref, xe_hbm.at[pl.ds(b * ACC, ACC)], sem).start()

    def wb_wait(b, acc_ref, sem):
        pltpu.make_async_copy(acc_ref, xe_hbm.at[pl.ds(b * ACC, ACC)], sem).wait()

    def compute(fv_v, e, rows_ref, acc_ref):
        @plsc.parallel_loop(0, ACC // 16, unroll=8)
        def _(i):
            acc_ref[pl.ds(16 * i, 16)] = zeros16f

        off = e * L

        @plsc.parallel_loop(0, L, unroll=4)
        def _(l):
            pos = jnp.full((16,), off + l, jnp.int32)
            fv = plsc.load_gather(fv_v, [pos])
            fs = fv & 31
            vs = lax.bitcast_convert_type(fv & ~31, jnp.float32)
            addr = fs * HID + iota16
            for j in range(HID // 16):
                x = vs * rows_ref[l, pl.ds(16 * j, 16)]
                plsc.addupdate_scatter(acc_ref, [addr + 16 * j], x)

    bufs = ((idx_0, fv_0, sem_i0), (idx_1, fv_1, sem_i1))

    def in_copies(c, bufset):
        idx_v, fv_v, sem_in = bufset
        cb = base + c * EPC
        return (
            pltpu.make_async_copy(
                index_hbm.at[pl.ds(cb * L, EPC * L)], idx_v, sem_in),
            pltpu.make_async_copy(
                fv_hbm.at[pl.ds(cb * L, EPC * L)], fv_v, sem_in),
        )

    n_chunks = n_per_w // EPC
    for cp in in_copies(0, bufs[0]):
        cp.start()

    for c in range(n_chunks):
        idx_v, fv_v, sem_in = bufs[c % 2]
        cb = base + c * EPC
        for cp in in_copies(c, bufs[c % 2]):
            cp.wait()
        if c + 1 < n_chunks:
            for cp in in_copies(c + 1, bufs[(c + 1) % 2]):
                cp.start()

        g_start(idx_v, 0, rows_a, sem_ga)

        def pair(ep, carry):
            e0 = 2 * ep
            e1 = 2 * ep + 1
            g_start(idx_v, e1, rows_b, sem_gb)

            @pl.when(ep > 0)
            def _():
                wb_wait(cb + e0 - 2, acc_a, sem_oa)
            g_wait(idx_v, e0, rows_a, sem_ga)
            compute(fv_v, e0, rows_a, acc_a)
            wb_start(cb + e0, acc_a, sem_oa)

            @pl.when(ep < EPC // 2 - 1)
            def _():
                g_start(idx_v, e1 + 1, rows_a, sem_ga)

            @pl.when(ep > 0)
            def _():
                wb_wait(cb + e1 - 2, acc_b, sem_ob)
            g_wait(idx_v, e1, rows_b, sem_gb)
            compute(fv_v, e1, rows_b, acc_b)
            wb_start(cb + e1, acc_b, sem_ob)
            return carry

        lax.fori_loop(0, EPC // 2, pair, 0)
        wb_wait(cb + EPC - 2, acc_a, sem_oa)
        wb_wait(cb + EPC - 1, acc_b, sem_ob)


def _sc_pool(index, fv, emb, nb):
    mesh = plsc.VectorSubcoreMesh(core_axis_name="c", subcore_axis_name="s")
    kern = pl.kernel(
        functools.partial(_sc_pool_kernel, nb),
        mesh=mesh,
        compiler_params=pltpu.CompilerParams(needs_layout_passes=False),
        out_type=jax.ShapeDtypeStruct((nb * ACC,), jnp.float32),
        scratch_types=[
            pltpu.VMEM((EPC * L,), jnp.int32),
            pltpu.VMEM((EPC * L,), jnp.int32),
            pltpu.VMEM((EPC * L,), jnp.int32),
            pltpu.VMEM((EPC * L,), jnp.int32),
            pltpu.VMEM((L, HID), jnp.float32),
            pltpu.VMEM((L, HID), jnp.float32),
            pltpu.VMEM((ACC,), jnp.float32),
            pltpu.VMEM((ACC,), jnp.float32),
            pltpu.SemaphoreType.DMA,
            pltpu.SemaphoreType.DMA,
            pltpu.SemaphoreType.DMA,
            pltpu.SemaphoreType.DMA,
            pltpu.SemaphoreType.DMA,
            pltpu.SemaphoreType.DMA,
        ],
    )
    return kern(index, fv, emb)


def _m_kernel(fe_ref, w1f_ref, m_ref):
    for s in range(NFIELDS):
        m_ref[s, :] = jnp.dot(fe_ref[s, :].reshape(1, HID), w1f_ref[s],
                              preferred_element_type=jnp.float32)[0]


def _mlp_kernel(xe_ref, val_ref, fld_ref, w1e_ref, m_ref, b1_ref, w2_ref,
                b2_ref, out_ref):
    h = jnp.dot(xe_ref[...].astype(jnp.bfloat16),
                w1e_ref[...].astype(jnp.bfloat16),
                preferred_element_type=jnp.float32)
    cols = []
    for s in range(VPAD):
        masked = jnp.where(fld_ref[...] == s, val_ref[...], 0.0)
        cols.append(jnp.sum(masked, axis=1, keepdims=True))
    vs = jnp.concatenate(cols, axis=1)
    h = h + jnp.dot(vs, m_ref[...], preferred_element_type=jnp.float32)
    h = jax.nn.relu(h + b1_ref[...])
    y = jnp.dot(h, w2_ref[...], preferred_element_type=jnp.float32) + b2_ref[0, 0]
    out_ref[...] = y


def kernel(index, value, field, emb, field_emb, W1, b1, W2, b2):
    index = index.astype(jnp.int32)
    field = field.astype(jnp.int32)

    vbits = lax.bitcast_convert_type(value, jnp.int32)
    fv = (vbits & ~31) | field
    index_f = index.reshape(-1)
    fv_f = fv.reshape(-1)
    emb2 = emb

    w1r = W1.reshape(NFIELDS, 2, HID, MLP_DIM)
    w1e = w1r[:, 0].reshape(NFIELDS * HID, MLP_DIM)
    w1f = w1r[:, 1]
    fe = field_emb[:NFIELDS]

    m = pl.pallas_call(
        _m_kernel,
        out_shape=jax.ShapeDtypeStruct((NFIELDS, MLP_DIM), jnp.float32),
    )(fe, w1f)
    m_pad = jnp.pad(m, ((0, VPAD - NFIELDS), (0, 0)))

    nh = 2
    nb = B // nh
    bt = 256
    outs = []
    for h in range(nh):
        xe_h = _sc_pool(index_f[h * nb * L:(h + 1) * nb * L],
                        fv_f[h * nb * L:(h + 1) * nb * L], emb2, nb)
        tile0 = h * (nb // bt)
        out_h = pl.pallas_call(
            _mlp_kernel,
            grid=(nb // bt,),
            in_specs=[
                pl.BlockSpec((bt, NFIELDS * HID), lambda i: (i, 0)),
                pl.BlockSpec((bt, L), lambda i, t=tile0: (i + t, 0)),
                pl.BlockSpec((bt, L), lambda i, t=tile0: (i + t, 0)),
                pl.BlockSpec((NFIELDS * HID, MLP_DIM), lambda i: (0, 0)),
                pl.BlockSpec((VPAD, MLP_DIM), lambda i: (0, 0)),
                pl.BlockSpec((1, MLP_DIM), lambda i: (0, 0)),
                pl.BlockSpec((MLP_DIM, 1), lambda i: (0, 0)),
                pl.BlockSpec((1, 1), lambda i: (0, 0)),
            ],
            out_specs=pl.BlockSpec((bt, 1), lambda i: (i, 0)),
            out_shape=jax.ShapeDtypeStruct((nb, 1), jnp.float32),
        )(xe_h.reshape(nb, NFIELDS * HID), value, field, w1e, m_pad,
          b1.reshape(1, MLP_DIM), W2, b2.reshape(1, 1))
        outs.append(out_h[:, 0])

    return jnp.concatenate(outs)


# speedup vs baseline: 1.0112x; 1.0008x over previous
import functools

import jax
import jax.numpy as jnp
from jax import lax
from jax.experimental import pallas as pl
from jax.experimental.pallas import tpu as pltpu
from jax.experimental.pallas import tpu_sc as plsc

B, L = 4096, 200
VOCAB = 100001
HID = 128
NFIELDS = 26
MLP_DIM = 512
VPAD = 32
NW = 32
EPC = 32
GC0, GC1 = 104, 96
ACC = NFIELDS * HID


def _sc_pool_kernel(nb, index_hbm, fv_hbm, emb_hbm,
                    xe_hbm,
                    idx_0, fv_0, idx_1, fv_1,
                    rows_a, rows_b, acc_a, acc_b,
                    sem_i0, sem_i1, sem_ga, sem_gb, sem_oa, sem_ob):
    nc = 2
    wid = lax.axis_index("s") * nc + lax.axis_index("c")
    n_per_w = nb // NW
    base = wid * n_per_w

    zeros16f = jnp.zeros((16,), jnp.float32)
    iota16 = lax.iota(jnp.int32, 16)

    def g_start(idx_v, e, rows_ref, sem):
        off = e * L
        pltpu.make_async_copy(
            emb_hbm.at[idx_v.at[pl.ds(off, GC0)]],
            rows_ref.at[pl.ds(0, GC0)], sem).start()
        pltpu.make_async_copy(
            emb_hbm.at[idx_v.at[pl.ds(off + GC0, GC1)]],
            rows_ref.at[pl.ds(GC0, GC1)], sem).start()

    def g_wait(idx_v, e, rows_ref, sem):
        off = e * L
        pltpu.make_async_copy(
            emb_hbm.at[idx_v.at[pl.ds(off, GC0)]],
            rows_ref.at[pl.ds(0, GC0)], sem).wait()
        pltpu.make_async_copy(
            emb_hbm.at[idx_v.at[pl.ds(off + GC0, GC1)]],
            rows_ref.at[pl.ds(GC0, GC1)], sem).wait()

    def wb_start(b, acc_ref, sem):
        pltpu.make_async_copy(acc_ref, xe_hbm.at[pl.ds(b * ACC, ACC)], sem).start()

    def wb_wait(b, acc_ref, sem):
        pltpu.make_async_copy(acc_ref, xe_hbm.at[pl.ds(b * ACC, ACC)], sem).wait()

    def compute(fv_v, e, rows_ref, acc_ref):
        @plsc.parallel_loop(0, ACC // 16, unroll=8)
        def _(i):
            acc_ref[pl.ds(16 * i, 16)] = zeros16f

        off = e * L

        @plsc.parallel_loop(0, L, unroll=8)
        def _(l):
            pos = jnp.full((16,), off + l, jnp.int32)
            fv = plsc.load_gather(fv_v, [pos])
            fs = fv & 31
            vs = lax.bitcast_convert_type(fv & ~31, jnp.float32)
            addr = fs * HID + iota16
            for j in range(HID // 16):
                x = vs * rows_ref[l, pl.ds(16 * j, 16)]
                plsc.addupdate_scatter(acc_ref, [addr + 16 * j], x)

    bufs = ((idx_0, fv_0, sem_i0), (idx_1, fv_1, sem_i1))

    def in_copies(c, bufset):
        idx_v, fv_v, sem_in = bufset
        cb = base + c * EPC
        return (
            pltpu.make_async_copy(
                index_hbm.at[pl.ds(cb * L, EPC * L)], idx_v, sem_in),
            pltpu.make_async_copy(
                fv_hbm.at[pl.ds(cb * L, EPC * L)], fv_v, sem_in),
        )

    n_chunks = n_per_w // EPC
    for cp in in_copies(0, bufs[0]):
        cp.start()

    for c in range(n_chunks):
        idx_v, fv_v, sem_in = bufs[c % 2]
        cb = base + c * EPC
        for cp in in_copies(c, bufs[c % 2]):
            cp.wait()
        if c + 1 < n_chunks:
            for cp in in_copies(c + 1, bufs[(c + 1) % 2]):
                cp.start()

        g_start(idx_v, 0, rows_a, sem_ga)

        def pair(ep, carry):
            e0 = 2 * ep
            e1 = 2 * ep + 1
            g_start(idx_v, e1, rows_b, sem_gb)

            @pl.when(ep > 0)
            def _():
                wb_wait(cb + e0 - 2, acc_a, sem_oa)
            g_wait(idx_v, e0, rows_a, sem_ga)
            compute(fv_v, e0, rows_a, acc_a)
            wb_start(cb + e0, acc_a, sem_oa)

            @pl.when(ep < EPC // 2 - 1)
            def _():
                g_start(idx_v, e1 + 1, rows_a, sem_ga)

            @pl.when(ep > 0)
            def _():
                wb_wait(cb + e1 - 2, acc_b, sem_ob)
            g_wait(idx_v, e1, rows_b, sem_gb)
            compute(fv_v, e1, rows_b, acc_b)
            wb_start(cb + e1, acc_b, sem_ob)
            return carry

        lax.fori_loop(0, EPC // 2, pair, 0)
        wb_wait(cb + EPC - 2, acc_a, sem_oa)
        wb_wait(cb + EPC - 1, acc_b, sem_ob)


def _sc_pool(index, fv, emb, nb):
    mesh = plsc.VectorSubcoreMesh(core_axis_name="c", subcore_axis_name="s")
    kern = pl.kernel(
        functools.partial(_sc_pool_kernel, nb),
        mesh=mesh,
        compiler_params=pltpu.CompilerParams(needs_layout_passes=False),
        out_type=jax.ShapeDtypeStruct((nb * ACC,), jnp.float32),
        scratch_types=[
            pltpu.VMEM((EPC * L,), jnp.int32),
            pltpu.VMEM((EPC * L,), jnp.int32),
            pltpu.VMEM((EPC * L,), jnp.int32),
            pltpu.VMEM((EPC * L,), jnp.int32),
            pltpu.VMEM((L, HID), jnp.float32),
            pltpu.VMEM((L, HID), jnp.float32),
            pltpu.VMEM((ACC,), jnp.float32),
            pltpu.VMEM((ACC,), jnp.float32),
            pltpu.SemaphoreType.DMA,
            pltpu.SemaphoreType.DMA,
            pltpu.SemaphoreType.DMA,
            pltpu.SemaphoreType.DMA,
            pltpu.SemaphoreType.DMA,
            pltpu.SemaphoreType.DMA,
        ],
    )
    return kern(index, fv, emb)


def _m_kernel(fe_ref, w1f_ref, m_ref):
    for s in range(NFIELDS):
        m_ref[s, :] = jnp.dot(fe_ref[s, :].reshape(1, HID), w1f_ref[s],
                              preferred_element_type=jnp.float32)[0]


def _mlp_kernel(xe_ref, val_ref, fld_ref, w1e_ref, m_ref, b1_ref, w2_ref,
                b2_ref, out_ref):
    h = jnp.dot(xe_ref[...].astype(jnp.bfloat16),
                w1e_ref[...].astype(jnp.bfloat16),
                preferred_element_type=jnp.float32)
    cols = []
    for s in range(VPAD):
        masked = jnp.where(fld_ref[...] == s, val_ref[...], 0.0)
        cols.append(jnp.sum(masked, axis=1, keepdims=True))
    vs = jnp.concatenate(cols, axis=1)
    h = h + jnp.dot(vs, m_ref[...], preferred_element_type=jnp.float32)
    h = jax.nn.relu(h + b1_ref[...])
    y = jnp.dot(h, w2_ref[...], preferred_element_type=jnp.float32) + b2_ref[0, 0]
    out_ref[...] = y


def kernel(index, value, field, emb, field_emb, W1, b1, W2, b2):
    index = index.astype(jnp.int32)
    field = field.astype(jnp.int32)

    vbits = lax.bitcast_convert_type(value, jnp.int32)
    fv = (vbits & ~31) | field
    index_f = index.reshape(-1)
    fv_f = fv.reshape(-1)
    emb2 = emb

    w1r = W1.reshape(NFIELDS, 2, HID, MLP_DIM)
    w1e = w1r[:, 0].reshape(NFIELDS * HID, MLP_DIM)
    w1f = w1r[:, 1]
    fe = field_emb[:NFIELDS]

    m = pl.pallas_call(
        _m_kernel,
        out_shape=jax.ShapeDtypeStruct((NFIELDS, MLP_DIM), jnp.float32),
    )(fe, w1f)
    m_pad = jnp.pad(m, ((0, VPAD - NFIELDS), (0, 0)))

    nh = 2
    nb = B // nh
    bt = 256
    outs = []
    for h in range(nh):
        xe_h = _sc_pool(index_f[h * nb * L:(h + 1) * nb * L],
                        fv_f[h * nb * L:(h + 1) * nb * L], emb2, nb)
        tile0 = h * (nb // bt)
        out_h = pl.pallas_call(
            _mlp_kernel,
            grid=(nb // bt,),
            in_specs=[
                pl.BlockSpec((bt, NFIELDS * HID), lambda i: (i, 0)),
                pl.BlockSpec((bt, L), lambda i, t=tile0: (i + t, 0)),
                pl.BlockSpec((bt, L), lambda i, t=tile0: (i + t, 0)),
                pl.BlockSpec((NFIELDS * HID, MLP_DIM), lambda i: (0, 0)),
                pl.BlockSpec((VPAD, MLP_DIM), lambda i: (0, 0)),
                pl.BlockSpec((1, MLP_DIM), lambda i: (0, 0)),
                pl.BlockSpec((MLP_DIM, 1), lambda i: (0, 0)),
                pl.BlockSpec((1, 1), lambda i: (0, 0)),
            ],
            out_specs=pl.BlockSpec((bt, 1), lambda i: (i, 0)),
            out_shape=jax.ShapeDtypeStruct((nb, 1), jnp.float32),
        )(xe_h.reshape(nb, NFIELDS * HID), value, field, w1e, m_pad,
          b1.reshape(1, MLP_DIM), W2, b2.reshape(1, 1))
        outs.append(out_h[:, 0])

    return jnp.concatenate(outs)
